# baseline (speedup 1.0000x reference)
import functools

import jax
import jax.numpy as jnp
from jax import lax
from jax.experimental import pallas as pl
from jax.experimental.pallas import tpu as pltpu
from jax.experimental.pallas import tpu_sc as plsc

_H = 4
_DH = 64


_C = 32
_ROWS_T = 360
_STAGE = 368
_PAD = 2176


def _cdiv(a, b):
    return (a + b - 1) // b


def _segsum_body(np_, has_attr, d,
                 table_h, attr_h, srcix_h, attrix_h, dst_h, meta_h, out_h,
                 meta_v, srcv, dstv, attrv, msgv, attrbuf, stage, sem):
    c = lax.axis_index("c")
    t = lax.axis_index("s")
    wid = 16 * c + t

    def _pass(p, _):
        moff = pl.multiple_of((p * 32 + wid) * 16, 8)
        pltpu.sync_copy(meta_h.at[pl.ds(moff, 16)], meta_v)
        mrow = meta_v[...]
        a = mrow[0]
        b = mrow[1]
        r0 = mrow[2]
        nrows = mrow[3]

        def _zstage(j, _):
            for k in range(d // 16):
                stage[j, pl.ds(16 * k, 16)] = jnp.zeros((16,), jnp.float32)
            return 0
        lax.fori_loop(0, _STAGE, _zstage, 0)

        nch = _cdiv(b - a, _C)

        def _chunk(i, _):
            e = pl.multiple_of(a + i * _C, 8)
            pltpu.sync_copy(srcix_h.at[pl.ds(e, _C)], srcv)
            pltpu.sync_copy(dst_h.at[pl.ds(e, _C)], dstv)
            pltpu.async_copy(table_h.at[srcv], msgv, sem).wait()
            if has_attr:
                pltpu.sync_copy(attrix_h.at[pl.ds(e, _C)], attrv)
                pltpu.async_copy(attr_h.at[attrv], attrbuf, sem).wait()

            rels = []
            for g in range(_C // 16):
                dgrp = dstv[pl.ds(16 * g, 16)]
                rel = dgrp - r0
                ok = (rel >= 0) & (rel < nrows)
                relv = jnp.where(ok, rel, _STAGE - 1)
                rels.extend([relv[j2] for j2 in range(16)])

            def _acc(k, _):
                cs = pl.ds(16 * k, 16)
                for j in range(_C):
                    v = stage[rels[j], cs] + msgv[j, cs]
                    if has_attr:
                        v = v + attrbuf[j, cs]
                    stage[rels[j], cs] = v
                return 0
            lax.fori_loop(0, d // 16, _acc, 0)
            return 0
        lax.fori_loop(0, nch, _chunk, 0)

        ng = nrows // 8

        def _wout(g, _):
            ga = pl.multiple_of(8 * g, 8)
            gb = pl.multiple_of(r0 + 8 * g, 8)
            pltpu.sync_copy(stage.at[pl.ds(ga, 8)], out_h.at[pl.ds(gb, 8)])
            return 0
        lax.fori_loop(0, ng, _wout, 0)
        return 0
    lax.fori_loop(0, np_, _pass, 0)


def _sc_segsum(table, attr, src_idx, attr_idx, dst_sorted, meta, s_out, np_):
    e_pad = src_idx.shape[0]
    d = table.shape[1]
    has_attr = attr is not None
    if not has_attr:
        attr = jnp.zeros((8, d), jnp.float32)
        attr_idx = jnp.zeros((e_pad,), jnp.int32)
    mesh = plsc.VectorSubcoreMesh(core_axis_name="c", subcore_axis_name="s")
    kern = pl.kernel(
        functools.partial(_segsum_body, np_, has_attr, d),
        out_type=jax.ShapeDtypeStruct((s_out, d), jnp.float32),
        mesh=mesh,
        scratch_types=[
            pltpu.VMEM((16,), jnp.int32),
            pltpu.VMEM((_C,), jnp.int32),
            pltpu.VMEM((_C,), jnp.int32),
            pltpu.VMEM((_C,), jnp.int32),
            pltpu.VMEM((_C, d), jnp.float32),
            pltpu.VMEM((_C, d), jnp.float32),
            pltpu.VMEM((_STAGE, d), jnp.float32),
            pltpu.SemaphoreType.DMA,
        ],
    )
    return kern(table, attr, src_idx, attr_idx, dst_sorted, meta)


def _sc_prep(dst, s_out):
    e = dst.shape[0]
    order = jnp.argsort(dst)
    dst_s = dst[order]
    e_pad = _cdiv(e, 128) * 128 + _PAD
    pad = e_pad - e
    dst_p = jnp.concatenate([dst_s, jnp.full((pad,), s_out, jnp.int32)])

    nu = _cdiv(s_out, _ROWS_T)
    np_ = _cdiv(nu, 32)
    n_units = 32 * np_
    r0s = jnp.minimum(jnp.arange(n_units, dtype=jnp.int32) * _ROWS_T, s_out)
    r1s = jnp.minimum(r0s + _ROWS_T, s_out)
    nrows = r1s - r0s
    a = (jnp.searchsorted(dst_s, r0s, side="left").astype(jnp.int32) // 8) * 8
    b = _cdiv(jnp.searchsorted(dst_s, r1s, side="left").astype(jnp.int32), 8) * 8
    meta = jnp.stack([a, b, r0s, nrows], axis=1)
    meta = jnp.pad(meta, ((0, 0), (0, 12))).reshape(-1).astype(jnp.int32)
    return order, dst_p, meta, np_, pad


def _sc_sorted_idx(values, order, pad):
    v = values[order]
    return jnp.concatenate([v, jnp.zeros((pad,), jnp.int32)]).astype(jnp.int32)


def _ln(x, g, b):
    mu = x.mean(-1, keepdims=True)
    v = ((x - mu) ** 2).mean(-1, keepdims=True)
    return (x - mu) / jnp.sqrt(v + 1e-5) * g + b


def _ffn_ln_body(relu_out, has_agg2, agg_ref, agg2_ref, h_ref, w1_ref,
                 b1_ref, w2_ref, b2_ref, g_ref, b_ref, o_ref):
    acc = agg_ref[...]
    if has_agg2:
        acc = acc + agg2_ref[...]
    z = jnp.maximum(acc @ w1_ref[...] + b1_ref[...], 0.0)
    z = z @ w2_ref[...] + b2_ref[...]
    y = _ln(h_ref[...] + z, g_ref[...], b_ref[...])
    if relu_out:
        y = jnp.maximum(y, 0.0)
    o_ref[...] = y


def _ffn_ln(agg, h, w1, b1, w2, b2, g, b, relu_out, agg2=None, bm=1000):
    m, d = agg.shape
    dh = w1.shape[1]
    has_agg2 = agg2 is not None
    if agg2 is None:
        agg2 = jnp.zeros((bm, d), jnp.float32)
        a2_spec = pl.BlockSpec((bm, d), lambda i: (0, 0))
    else:
        a2_spec = pl.BlockSpec((bm, d), lambda i: (i, 0))
    b1 = b1.reshape(1, dh)
    b2 = b2.reshape(1, d)
    g = g.reshape(1, d)
    b = b.reshape(1, d)
    return pl.pallas_call(
        functools.partial(_ffn_ln_body, relu_out, has_agg2),
        grid=(m // bm,),
        in_specs=[
            pl.BlockSpec((bm, d), lambda i: (i, 0)),
            a2_spec,
            pl.BlockSpec((bm, d), lambda i: (i, 0)),
            pl.BlockSpec((d, dh), lambda i: (0, 0)),
            pl.BlockSpec((1, dh), lambda i: (0, 0)),
            pl.BlockSpec((dh, d), lambda i: (0, 0)),
            pl.BlockSpec((1, d), lambda i: (0, 0)),
            pl.BlockSpec((1, d), lambda i: (0, 0)),
            pl.BlockSpec((1, d), lambda i: (0, 0)),
        ],
        out_specs=pl.BlockSpec((bm, d), lambda i: (i, 0)),
        out_shape=jax.ShapeDtypeStruct((m, d), jnp.float32),
    )(agg, agg2, h, w1, b1, w2, b2, g, b)


def _pe_update_body(pe_ref, agg_ref, w_ref, o_ref):
    o_ref[...] = pe_ref[...] + jnp.maximum(agg_ref[...] @ w_ref[...], 0.0)


def _pe_update(pe, agg, w, bm=1000):
    m, d = pe.shape
    return pl.pallas_call(
        _pe_update_body,
        grid=(m // bm,),
        in_specs=[
            pl.BlockSpec((bm, d), lambda i: (i, 0)),
            pl.BlockSpec((bm, d), lambda i: (i, 0)),
            pl.BlockSpec((d, d), lambda i: (0, 0)),
        ],
        out_specs=pl.BlockSpec((bm, d), lambda i: (i, 0)),
        out_shape=jax.ShapeDtypeStruct((m, d), jnp.float32),
    )(pe, agg, w)


def _matmul_body(x_ref, w_ref, o_ref):
    o_ref[...] = x_ref[...] @ w_ref[...]


def _matmul(x, w, bm=1000):
    m, d = x.shape
    k = w.shape[1]
    return pl.pallas_call(
        _matmul_body,
        grid=(m // bm,),
        in_specs=[
            pl.BlockSpec((bm, d), lambda i: (i, 0)),
            pl.BlockSpec((d, k), lambda i: (0, 0)),
        ],
        out_specs=pl.BlockSpec((bm, k), lambda i: (i, 0)),
        out_shape=jax.ShapeDtypeStruct((m, k), jnp.float32),
    )(x, w)


def _proj_ln_body(x_ref, att_ref, wo_ref, g_ref, b_ref, o_ref):
    o_ref[...] = _ln(x_ref[...] + att_ref[...] @ wo_ref[...],
                     g_ref[...], b_ref[...])


def _proj_ln(x, att, wo, g, b, bm=1000):
    m, d = x.shape
    g = g.reshape(1, d)
    b = b.reshape(1, d)
    return pl.pallas_call(
        _proj_ln_body,
        grid=(m // bm,),
        in_specs=[
            pl.BlockSpec((bm, d), lambda i: (i, 0)),
            pl.BlockSpec((bm, d), lambda i: (i, 0)),
            pl.BlockSpec((d, d), lambda i: (0, 0)),
            pl.BlockSpec((1, d), lambda i: (0, 0)),
            pl.BlockSpec((1, d), lambda i: (0, 0)),
        ],
        out_specs=pl.BlockSpec((bm, d), lambda i: (i, 0)),
        out_shape=jax.ShapeDtypeStruct((m, d), jnp.float32),
    )(x, att, wo, g, b)


def kernel(x_ab, edge_index_ab, edge_attr_ab, pe_ab, batch_list, x_ba,
           edge_index_ba, edge_attr_ba, edge_map_ab, params):
    n, d = x_ab.shape
    e_ab = edge_attr_ab.shape[0]
    ng = 128
    layers = len(params["ab"])

    src_ab, dst_ab = edge_index_ab[0], edge_index_ab[1]
    src_ba, dst_ba = edge_index_ba[0], edge_index_ba[1]

    order_ab, dstp_ab, meta_ab, np_ab, pad_ab = _sc_prep(dst_ab, n)
    order_ba, dstp_ba, meta_ba, np_ba, pad_ba = _sc_prep(dst_ba, e_ab)
    srcs_ab = _sc_sorted_idx(src_ab, order_ab, pad_ab)
    srcs_ba = _sc_sorted_idx(src_ba, order_ba, pad_ba)
    attr0_idx = _sc_sorted_idx(jnp.arange(e_ab, dtype=jnp.int32),
                               order_ab, pad_ab)
    emap_idx = _sc_sorted_idx(edge_map_ab, order_ab, pad_ab)
    attrba_idx = _sc_sorted_idx(jnp.arange(e_ab, dtype=jnp.int32),
                                order_ba, pad_ba)
    attr_agg_ba = _sc_segsum(edge_attr_ba, None, attrba_idx, None,
                             dstp_ba, meta_ba, e_ab, np_ba)

    h_ab = x_ab
    pe = pe_ab
    h_ba = x_ba

    for l in range(layers):
        last_act = (l != layers - 1)
        p = params["ab"][l]
        if l == 0:
            agg = _sc_segsum(h_ab, edge_attr_ab, srcs_ab, attr0_idx,
                             dstp_ab, meta_ab, n, np_ab)
        else:
            agg = _sc_segsum(h_ab, h_ba, srcs_ab, emap_idx,
                             dstp_ab, meta_ab, n, np_ab)
        h_new = _ffn_ln(agg, h_ab, p["W1"], p["b1"], p["W2"], p["b2"],
                        p["ln_g"], p["ln_b"], relu_out=last_act)
        pe_agg = _sc_segsum(pe, None, srcs_ab, None,
                            dstp_ab, meta_ab, n, np_ab)
        pe = _pe_update(pe, pe_agg, p["Wpe"])
        h_ab = h_new

        q = params["ba"][l]
        agg_b = _sc_segsum(h_ba, None, srcs_ba, None,
                           dstp_ba, meta_ba, e_ab, np_ba)
        h_ba = _ffn_ln(agg_b, h_ba, q["W1"], q["b1"], q["W2"], q["b2"],
                       q["ln_g"], q["ln_b"], relu_out=last_act,
                       agg2=attr_agg_ba)

    node_repr = h_ab
    x = node_repr + _matmul(pe, params["tr_pe_in"])
    for t in params["tr"]:
        wqkv = jnp.concatenate([t["Wq"], t["Wk"], t["Wv"]], axis=1)
        qkv = _matmul(x, wqkv)
        qh = qkv[:, :d].reshape(n, _H, _DH)
        kh = qkv[:, d:2 * d].reshape(n, _H, _DH)
        vh = qkv[:, 2 * d:].reshape(n, _H, _DH)
        sc = (qh[dst_ab] * kh[src_ab]).sum(-1) / jnp.sqrt(jnp.float32(_DH))
        m = jax.ops.segment_max(sc, dst_ab, num_segments=n)
        ex = jnp.exp(sc - m[dst_ab])
        s = jax.ops.segment_sum(ex, dst_ab, num_segments=n)
        alpha = ex / (s[dst_ab] + 1e-9)
        att = jax.ops.segment_sum(alpha[:, :, None] * vh[src_ab], dst_ab,
                                  num_segments=n)
        x = _proj_ln(x, att.reshape(n, d), t["Wo"], t["ln1_g"], t["ln1_b"])
        x = _ffn_ln(x, x, t["ffW1"], t["ffb1"], t["ffW2"], t["ffb2"],
                    t["ln2_g"], t["ln2_b"], relu_out=False)

    node_feat = node_repr + x
    pe_lin = _matmul(pe, jnp.pad(params["pe_out_W"], ((0, 0), (0, 112)))
                     )[:, :16] + params["pe_out_b"]
    pe_repr = pe_lin / (jnp.linalg.norm(pe_lin, axis=0, keepdims=True) + 1e-9)
    sums = jax.ops.segment_sum(node_feat, batch_list, num_segments=ng)
    counts = jax.ops.segment_sum(jnp.ones((n,), jnp.float32), batch_list,
                                 num_segments=ng)
    graph_repr = sums / jnp.maximum(counts, 1.0)[:, None]
    return graph_repr, pe_repr, batch_list


# speedup vs baseline: 1.0342x; 1.0342x over previous
import functools

import jax
import jax.numpy as jnp
from jax import lax
from jax.experimental import pallas as pl
from jax.experimental.pallas import tpu as pltpu
from jax.experimental.pallas import tpu_sc as plsc

_H = 4
_DH = 64


_C = 32
_ROWS_T = 360
_STAGE = 368
_PAD = 2176


def _cdiv(a, b):
    return (a + b - 1) // b


def _segsum_body(np_, has_attr, d,
                 table_h, attr_h, srcix_h, attrix_h, dst_h, meta_h, out_h,
                 meta_v, srcv, dstv, attrv, msgv, attrbuf, stage, sem):
    c = lax.axis_index("c")
    t = lax.axis_index("s")
    wid = 16 * c + t

    def _pass(p, _):
        moff = pl.multiple_of((p * 32 + wid) * 16, 8)
        pltpu.sync_copy(meta_h.at[pl.ds(moff, 16)], meta_v)
        mrow = meta_v[...]
        a = mrow[0]
        b = mrow[1]
        r0 = mrow[2]
        nrows = mrow[3]

        def _zstage(j, _):
            for k in range(d // 16):
                stage[j, pl.ds(16 * k, 16)] = jnp.zeros((16,), jnp.float32)
            return 0
        lax.fori_loop(0, _STAGE, _zstage, 0)

        nch = _cdiv(b - a, _C)

        def _chunk(i, _):
            e = pl.multiple_of(a + i * _C, 8)
            pltpu.sync_copy(srcix_h.at[pl.ds(e, _C)], srcv)
            pltpu.sync_copy(dst_h.at[pl.ds(e, _C)], dstv)
            pltpu.async_copy(table_h.at[srcv], msgv, sem).wait()
            if has_attr:
                pltpu.sync_copy(attrix_h.at[pl.ds(e, _C)], attrv)
                pltpu.async_copy(attr_h.at[attrv], attrbuf, sem).wait()

            rels = []
            for g in range(_C // 16):
                dgrp = dstv[pl.ds(16 * g, 16)]
                rel = dgrp - r0
                ok = (rel >= 0) & (rel < nrows)
                relv = jnp.where(ok, rel, _STAGE - 1)
                rels.extend([relv[j2] for j2 in range(16)])

            def _acc(k, _):
                cs = pl.ds(16 * k, 16)
                for j in range(_C):
                    v = stage[rels[j], cs] + msgv[j, cs]
                    if has_attr:
                        v = v + attrbuf[j, cs]
                    stage[rels[j], cs] = v
                return 0
            lax.fori_loop(0, d // 16, _acc, 0)
            return 0
        lax.fori_loop(0, nch, _chunk, 0)

        ng = nrows // 8

        def _wout(g, _):
            ga = pl.multiple_of(8 * g, 8)
            gb = pl.multiple_of(r0 + 8 * g, 8)
            pltpu.sync_copy(stage.at[pl.ds(ga, 8)], out_h.at[pl.ds(gb, 8)])
            return 0
        lax.fori_loop(0, ng, _wout, 0)
        return 0
    lax.fori_loop(0, np_, _pass, 0)


def _sc_segsum(table, attr, src_idx, attr_idx, dst_sorted, meta, s_out, np_):
    e_pad = src_idx.shape[0]
    d = table.shape[1]
    has_attr = attr is not None
    if not has_attr:
        attr = jnp.zeros((8, d), jnp.float32)
        attr_idx = jnp.zeros((e_pad,), jnp.int32)
    mesh = plsc.VectorSubcoreMesh(core_axis_name="c", subcore_axis_name="s")
    kern = pl.kernel(
        functools.partial(_segsum_body, np_, has_attr, d),
        out_type=jax.ShapeDtypeStruct((s_out, d), jnp.float32),
        mesh=mesh,
        scratch_types=[
            pltpu.VMEM((16,), jnp.int32),
            pltpu.VMEM((_C,), jnp.int32),
            pltpu.VMEM((_C,), jnp.int32),
            pltpu.VMEM((_C,), jnp.int32),
            pltpu.VMEM((_C, d), jnp.float32),
            pltpu.VMEM((_C, d), jnp.float32),
            pltpu.VMEM((_STAGE, d), jnp.float32),
            pltpu.SemaphoreType.DMA,
        ],
    )
    return kern(table, attr, src_idx, attr_idx, dst_sorted, meta)


def _sc_prep(dst, s_out):
    e = dst.shape[0]
    order = jnp.argsort(dst)
    dst_s = dst[order]
    e_pad = _cdiv(e, 128) * 128 + _PAD
    pad = e_pad - e
    dst_p = jnp.concatenate([dst_s, jnp.full((pad,), s_out, jnp.int32)])

    nu = _cdiv(s_out, _ROWS_T)
    np_ = _cdiv(nu, 32)
    n_units = 32 * np_
    r0s = jnp.minimum(jnp.arange(n_units, dtype=jnp.int32) * _ROWS_T, s_out)
    r1s = jnp.minimum(r0s + _ROWS_T, s_out)
    nrows = r1s - r0s
    a = (jnp.searchsorted(dst_s, r0s, side="left").astype(jnp.int32) // 8) * 8
    b = _cdiv(jnp.searchsorted(dst_s, r1s, side="left").astype(jnp.int32), 8) * 8
    meta = jnp.stack([a, b, r0s, nrows], axis=1)
    meta = jnp.pad(meta, ((0, 0), (0, 12))).reshape(-1).astype(jnp.int32)
    return order, dst_p, meta, np_, pad


def _sc_sorted_idx(values, order, pad):
    v = values[order]
    return jnp.concatenate([v, jnp.zeros((pad,), jnp.int32)]).astype(jnp.int32)


def _dot(a, b):
    return lax.dot_general(a, b, (((1,), (0,)), ((), ())),
                           precision=lax.Precision.DEFAULT,
                           preferred_element_type=jnp.float32)


def _ln(x, g, b):
    mu = x.mean(-1, keepdims=True)
    v = ((x - mu) ** 2).mean(-1, keepdims=True)
    return (x - mu) / jnp.sqrt(v + 1e-5) * g + b


def _ffn_ln_body(relu_out, has_agg2, agg_ref, agg2_ref, h_ref, w1_ref,
                 b1_ref, w2_ref, b2_ref, g_ref, b_ref, o_ref):
    acc = agg_ref[...]
    if has_agg2:
        acc = acc + agg2_ref[...]
    z = jnp.maximum(_dot(acc, w1_ref[...]) + b1_ref[...], 0.0)
    z = _dot(z, w2_ref[...]) + b2_ref[...]
    y = _ln(h_ref[...] + z, g_ref[...], b_ref[...])
    if relu_out:
        y = jnp.maximum(y, 0.0)
    o_ref[...] = y


def _ffn_ln(agg, h, w1, b1, w2, b2, g, b, relu_out, agg2=None, bm=1000):
    m, d = agg.shape
    dh = w1.shape[1]
    has_agg2 = agg2 is not None
    if agg2 is None:
        agg2 = jnp.zeros((bm, d), jnp.float32)
        a2_spec = pl.BlockSpec((bm, d), lambda i: (0, 0))
    else:
        a2_spec = pl.BlockSpec((bm, d), lambda i: (i, 0))
    b1 = b1.reshape(1, dh)
    b2 = b2.reshape(1, d)
    g = g.reshape(1, d)
    b = b.reshape(1, d)
    return pl.pallas_call(
        functools.partial(_ffn_ln_body, relu_out, has_agg2),
        grid=(m // bm,),
        in_specs=[
            pl.BlockSpec((bm, d), lambda i: (i, 0)),
            a2_spec,
            pl.BlockSpec((bm, d), lambda i: (i, 0)),
            pl.BlockSpec((d, dh), lambda i: (0, 0)),
            pl.BlockSpec((1, dh), lambda i: (0, 0)),
            pl.BlockSpec((dh, d), lambda i: (0, 0)),
            pl.BlockSpec((1, d), lambda i: (0, 0)),
            pl.BlockSpec((1, d), lambda i: (0, 0)),
            pl.BlockSpec((1, d), lambda i: (0, 0)),
        ],
        out_specs=pl.BlockSpec((bm, d), lambda i: (i, 0)),
        out_shape=jax.ShapeDtypeStruct((m, d), jnp.float32),
    )(agg, agg2, h, w1, b1, w2, b2, g, b)


def _pe_update_body(pe_ref, agg_ref, w_ref, o_ref):
    o_ref[...] = pe_ref[...] + jnp.maximum(_dot(agg_ref[...], w_ref[...]), 0.0)


def _pe_update(pe, agg, w, bm=1000):
    m, d = pe.shape
    return pl.pallas_call(
        _pe_update_body,
        grid=(m // bm,),
        in_specs=[
            pl.BlockSpec((bm, d), lambda i: (i, 0)),
            pl.BlockSpec((bm, d), lambda i: (i, 0)),
            pl.BlockSpec((d, d), lambda i: (0, 0)),
        ],
        out_specs=pl.BlockSpec((bm, d), lambda i: (i, 0)),
        out_shape=jax.ShapeDtypeStruct((m, d), jnp.float32),
    )(pe, agg, w)


def _matmul_body(x_ref, w_ref, o_ref):
    o_ref[...] = _dot(x_ref[...], w_ref[...])


def _matmul(x, w, bm=1000):
    m, d = x.shape
    k = w.shape[1]
    return pl.pallas_call(
        _matmul_body,
        grid=(m // bm,),
        in_specs=[
            pl.BlockSpec((bm, d), lambda i: (i, 0)),
            pl.BlockSpec((d, k), lambda i: (0, 0)),
        ],
        out_specs=pl.BlockSpec((bm, k), lambda i: (i, 0)),
        out_shape=jax.ShapeDtypeStruct((m, k), jnp.float32),
    )(x, w)


def _proj_ln_body(x_ref, att_ref, wo_ref, g_ref, b_ref, o_ref):
    o_ref[...] = _ln(x_ref[...] + _dot(att_ref[...], wo_ref[...]),
                     g_ref[...], b_ref[...])


def _proj_ln(x, att, wo, g, b, bm=1000):
    m, d = x.shape
    g = g.reshape(1, d)
    b = b.reshape(1, d)
    return pl.pallas_call(
        _proj_ln_body,
        grid=(m // bm,),
        in_specs=[
            pl.BlockSpec((bm, d), lambda i: (i, 0)),
            pl.BlockSpec((bm, d), lambda i: (i, 0)),
            pl.BlockSpec((d, d), lambda i: (0, 0)),
            pl.BlockSpec((1, d), lambda i: (0, 0)),
            pl.BlockSpec((1, d), lambda i: (0, 0)),
        ],
        out_specs=pl.BlockSpec((bm, d), lambda i: (i, 0)),
        out_shape=jax.ShapeDtypeStruct((m, d), jnp.float32),
    )(x, att, wo, g, b)


def kernel(x_ab, edge_index_ab, edge_attr_ab, pe_ab, batch_list, x_ba,
           edge_index_ba, edge_attr_ba, edge_map_ab, params):
    n, d = x_ab.shape
    e_ab = edge_attr_ab.shape[0]
    ng = 128
    layers = len(params["ab"])

    src_ab, dst_ab = edge_index_ab[0], edge_index_ab[1]
    src_ba, dst_ba = edge_index_ba[0], edge_index_ba[1]

    attr_agg_ba = jax.ops.segment_sum(edge_attr_ba, dst_ba, num_segments=e_ab)

    h_ab = x_ab
    e_attr = edge_attr_ab
    pe = pe_ab
    h_ba = x_ba

    for l in range(layers):
        last_act = (l != layers - 1)
        p = params["ab"][l]
        msg = h_ab[src_ab] + e_attr
        agg = jax.ops.segment_sum(msg, dst_ab, num_segments=n)
        h_new = _ffn_ln(agg, h_ab, p["W1"], p["b1"], p["W2"], p["b2"],
                        p["ln_g"], p["ln_b"], relu_out=last_act)
        pe_agg = jax.ops.segment_sum(pe[src_ab], dst_ab, num_segments=n)
        pe = _pe_update(pe, pe_agg, p["Wpe"])
        h_ab = h_new

        q = params["ba"][l]
        agg_b = jax.ops.segment_sum(h_ba[src_ba], dst_ba, num_segments=e_ab)
        h_ba = _ffn_ln(agg_b, h_ba, q["W1"], q["b1"], q["W2"], q["b2"],
                       q["ln_g"], q["ln_b"], relu_out=last_act,
                       agg2=attr_agg_ba)
        e_attr = h_ba[edge_map_ab]

    node_repr = h_ab
    x = node_repr + _matmul(pe, params["tr_pe_in"])
    for t in params["tr"]:
        wqkv = jnp.concatenate([t["Wq"], t["Wk"], t["Wv"]], axis=1)
        qkv = _matmul(x, wqkv)
        qh = qkv[:, :d].reshape(n, _H, _DH)
        kh = qkv[:, d:2 * d].reshape(n, _H, _DH)
        vh = qkv[:, 2 * d:].reshape(n, _H, _DH)
        sc = (qh[dst_ab] * kh[src_ab]).sum(-1) / jnp.sqrt(jnp.float32(_DH))
        m = jax.ops.segment_max(sc, dst_ab, num_segments=n)
        ex = jnp.exp(sc - m[dst_ab])
        s = jax.ops.segment_sum(ex, dst_ab, num_segments=n)
        alpha = ex / (s[dst_ab] + 1e-9)
        att = jax.ops.segment_sum(alpha[:, :, None] * vh[src_ab], dst_ab,
                                  num_segments=n)
        x = _proj_ln(x, att.reshape(n, d), t["Wo"], t["ln1_g"], t["ln1_b"])
        x = _ffn_ln(x, x, t["ffW1"], t["ffb1"], t["ffW2"], t["ffb2"],
                    t["ln2_g"], t["ln2_b"], relu_out=False)

    node_feat = node_repr + x
    pe_lin = _matmul(pe, jnp.pad(params["pe_out_W"], ((0, 0), (0, 112)))
                     )[:, :16] + params["pe_out_b"]
    pe_repr = pe_lin / (jnp.linalg.norm(pe_lin, axis=0, keepdims=True) + 1e-9)
    sums = jax.ops.segment_sum(node_feat, batch_list, num_segments=ng)
    counts = jax.ops.segment_sum(jnp.ones((n,), jnp.float32), batch_list,
                                 num_segments=ng)
    graph_repr = sums / jnp.maximum(counts, 1.0)[:, None]
    return graph_repr, pe_repr, batch_list


# speedup vs baseline: 1.2034x; 1.1636x over previous
import functools

import jax
import jax.numpy as jnp
from jax import lax
from jax.experimental import pallas as pl
from jax.experimental.pallas import tpu as pltpu

_H = 4
_DH = 64


def _dot(a, b):
    return lax.dot_general(a, b, (((1,), (0,)), ((), ())),
                           precision=lax.Precision.DEFAULT,
                           preferred_element_type=jnp.float32)


def _ln(x, g, b):
    mu = x.mean(-1, keepdims=True)
    v = ((x - mu) ** 2).mean(-1, keepdims=True)
    return (x - mu) / jnp.sqrt(v + 1e-5) * g + b


def _ffn_ln_body(relu_out, agg_ref, h_ref, w1_ref, b1_ref, w2_ref, b2_ref,
                 g_ref, b_ref, o_ref):
    z = jnp.maximum(_dot(agg_ref[...], w1_ref[...]) + b1_ref[...], 0.0)
    z = _dot(z, w2_ref[...]) + b2_ref[...]
    y = _ln(h_ref[...] + z, g_ref[...], b_ref[...])
    if relu_out:
        y = jnp.maximum(y, 0.0)
    o_ref[...] = y


def _ffn_ln(agg, h, w1, b1, w2, b2, g, b, relu_out, bm=1000):
    m, d = agg.shape
    dh = w1.shape[1]
    b1 = b1.reshape(1, dh)
    b2 = b2.reshape(1, d)
    g = g.reshape(1, d)
    b = b.reshape(1, d)
    return pl.pallas_call(
        functools.partial(_ffn_ln_body, relu_out),
        grid=(m // bm,),
        in_specs=[
            pl.BlockSpec((bm, d), lambda i: (i, 0)),
            pl.BlockSpec((bm, d), lambda i: (i, 0)),
            pl.BlockSpec((d, dh), lambda i: (0, 0)),
            pl.BlockSpec((1, dh), lambda i: (0, 0)),
            pl.BlockSpec((dh, d), lambda i: (0, 0)),
            pl.BlockSpec((1, d), lambda i: (0, 0)),
            pl.BlockSpec((1, d), lambda i: (0, 0)),
            pl.BlockSpec((1, d), lambda i: (0, 0)),
        ],
        out_specs=pl.BlockSpec((bm, d), lambda i: (i, 0)),
        out_shape=jax.ShapeDtypeStruct((m, d), jnp.float32),
    )(agg, h, w1, b1, w2, b2, g, b)


def _pe_update_body(pe_ref, agg_ref, w_ref, o_ref):
    o_ref[...] = pe_ref[...] + jnp.maximum(_dot(agg_ref[...], w_ref[...]), 0.0)


def _pe_update(pe, agg, w, bm=1000):
    m, d = pe.shape
    return pl.pallas_call(
        _pe_update_body,
        grid=(m // bm,),
        in_specs=[
            pl.BlockSpec((bm, d), lambda i: (i, 0)),
            pl.BlockSpec((bm, d), lambda i: (i, 0)),
            pl.BlockSpec((d, d), lambda i: (0, 0)),
        ],
        out_specs=pl.BlockSpec((bm, d), lambda i: (i, 0)),
        out_shape=jax.ShapeDtypeStruct((m, d), jnp.float32),
    )(pe, agg, w)


def _matmul_body(x_ref, w_ref, o_ref):
    o_ref[...] = _dot(x_ref[...], w_ref[...])


def _matmul(x, w, bm=1000):
    m, d = x.shape
    k = w.shape[1]
    return pl.pallas_call(
        _matmul_body,
        grid=(m // bm,),
        in_specs=[
            pl.BlockSpec((bm, d), lambda i: (i, 0)),
            pl.BlockSpec((d, k), lambda i: (0, 0)),
        ],
        out_specs=pl.BlockSpec((bm, k), lambda i: (i, 0)),
        out_shape=jax.ShapeDtypeStruct((m, k), jnp.float32),
    )(x, w)


def _proj_ln_body(x_ref, att_ref, wo_ref, g_ref, b_ref, o_ref):
    o_ref[...] = _ln(x_ref[...] + _dot(att_ref[...], wo_ref[...]),
                     g_ref[...], b_ref[...])


def _proj_ln(x, att, wo, g, b, bm=1000):
    m, d = x.shape
    g = g.reshape(1, d)
    b = b.reshape(1, d)
    return pl.pallas_call(
        _proj_ln_body,
        grid=(m // bm,),
        in_specs=[
            pl.BlockSpec((bm, d), lambda i: (i, 0)),
            pl.BlockSpec((bm, d), lambda i: (i, 0)),
            pl.BlockSpec((d, d), lambda i: (0, 0)),
            pl.BlockSpec((1, d), lambda i: (0, 0)),
            pl.BlockSpec((1, d), lambda i: (0, 0)),
        ],
        out_specs=pl.BlockSpec((bm, d), lambda i: (i, 0)),
        out_shape=jax.ShapeDtypeStruct((m, d), jnp.float32),
    )(x, att, wo, g, b)


def kernel(x_ab, edge_index_ab, edge_attr_ab, pe_ab, batch_list, x_ba,
           edge_index_ba, edge_attr_ba, edge_map_ab, params):
    n, d = x_ab.shape
    e_ab = edge_attr_ab.shape[0]
    ng = 128
    layers = len(params["ab"])

    src_ab, dst_ab = edge_index_ab[0], edge_index_ab[1]
    src_ba, dst_ba = edge_index_ba[0], edge_index_ba[1]

    h_ab = x_ab
    e_attr = edge_attr_ab
    pe = pe_ab
    h_ba = x_ba

    for l in range(layers):
        last_act = (l != layers - 1)
        p = params["ab"][l]
        msg = h_ab[src_ab] + e_attr
        agg = jax.ops.segment_sum(msg, dst_ab, num_segments=n)
        h_new = _ffn_ln(agg, h_ab, p["W1"], p["b1"], p["W2"], p["b2"],
                        p["ln_g"], p["ln_b"], relu_out=last_act)
        pe_agg = jax.ops.segment_sum(pe[src_ab], dst_ab, num_segments=n)
        pe = _pe_update(pe, pe_agg, p["Wpe"])
        h_ab = h_new

        q = params["ba"][l]
        msg_b = h_ba[src_ba] + edge_attr_ba
        agg_b = jax.ops.segment_sum(msg_b, dst_ba, num_segments=e_ab)
        h_ba = _ffn_ln(agg_b, h_ba, q["W1"], q["b1"], q["W2"], q["b2"],
                       q["ln_g"], q["ln_b"], relu_out=last_act)
        e_attr = h_ba[edge_map_ab]

    node_repr = h_ab
    x = node_repr + _matmul(pe, params["tr_pe_in"])
    for t in params["tr"]:
        wqkv = jnp.concatenate([t["Wq"], t["Wk"], t["Wv"]], axis=1)
        qkv = _matmul(x, wqkv)
        qh = qkv[:, :d].reshape(n, _H, _DH)
        kh = qkv[:, d:2 * d].reshape(n, _H, _DH)
        vh = qkv[:, 2 * d:].reshape(n, _H, _DH)
        sc = (qh[dst_ab] * kh[src_ab]).sum(-1) / jnp.sqrt(jnp.float32(_DH))
        m = jax.ops.segment_max(sc, dst_ab, num_segments=n)
        ex = jnp.exp(sc - m[dst_ab])
        s = jax.ops.segment_sum(ex, dst_ab, num_segments=n)
        alpha = ex / (s[dst_ab] + 1e-9)
        att = jax.ops.segment_sum(alpha[:, :, None] * vh[src_ab], dst_ab,
                                  num_segments=n)
        x = _proj_ln(x, att.reshape(n, d), t["Wo"], t["ln1_g"], t["ln1_b"])
        x = _ffn_ln(x, x, t["ffW1"], t["ffb1"], t["ffW2"], t["ffb2"],
                    t["ln2_g"], t["ln2_b"], relu_out=False)

    node_feat = node_repr + x
    pe_lin = _matmul(pe, jnp.pad(params["pe_out_W"], ((0, 0), (0, 112)))
                     )[:, :16] + params["pe_out_b"]
    pe_repr = pe_lin / (jnp.linalg.norm(pe_lin, axis=0, keepdims=True) + 1e-9)
    sums = jax.ops.segment_sum(node_feat, batch_list, num_segments=ng)
    counts = jax.ops.segment_sum(jnp.ones((n,), jnp.float32), batch_list,
                                 num_segments=ng)
    graph_repr = sums / jnp.maximum(counts, 1.0)[:, None]
    return graph_repr, pe_repr, batch_list
